# baseline (speedup 1.0000x reference)
import jax
import jax.numpy as jnp
from jax.experimental import pallas as pl
from jax.experimental.pallas import tpu as pltpu

NODES = 4096
HID = 32
NB = 4
CIN = HID + 1
WID = NB * CIN
SWID = NB * HID
R = 128
NBLK = NODES // R
F8 = jnp.float8_e4m3fn
SCALE = 256.0
INV = 1.0 / SCALE


def _mega_kernel(a0_ref, a1_ref, y_ref, s_ref, x_ref, wgx_ref, wgs_ref,
                 bg_ref, wux_ref, wus_ref, bu_ref, h_ref,
                 a0s, a1s, t0s, t1s, u0s, cbs, rs, xds):
    p = pl.program_id(0)
    i = pl.program_id(1)
    rows = pl.ds(i * R, R)
    f32 = jnp.float32
    bf16 = jnp.bfloat16


    def first_pass(a_ref, as_scratch, ts_scratch, xd_col):
        a = a_ref[...]
        as_scratch[rows, :] = (a * SCALE).astype(F8)
        t = jnp.dot(a.astype(bf16), y_ref[...], preferred_element_type=f32)
        ts_scratch[rows, :] = t[:, NB:].astype(bf16)
        xds[rows, xd_col:xd_col + NB] = t[:, :NB].astype(bf16)

    @pl.when(p == 0)
    def _():
        first_pass(a0_ref, a0s, t0s, 0)

    @pl.when(p == 1)
    def _():
        ab = a0s[rows, :].astype(bf16)
        u0 = jnp.dot(ab, t0s[...], preferred_element_type=f32) * INV
        u0s[rows, :] = u0.astype(bf16)
        xu = jnp.dot(ab, xds[:, 0:NB], preferred_element_type=f32) * INV
        xds[rows, NB:2 * NB] = xu.astype(bf16)

    @pl.when(p == 2)
    def _():
        first_pass(a1_ref, a1s, t1s, 2 * NB)

    @pl.when(p == 3)
    def _():
        ab = a1s[rows, :].astype(bf16)
        u1 = jnp.dot(ab, t1s[...], preferred_element_type=f32) * INV
        xu1 = jnp.dot(ab, xds[:, 2 * NB:3 * NB],
                      preferred_element_type=f32) * INV
        xds[rows, 3 * NB:4 * NB] = xu1.astype(bf16)
        xd = jnp.concatenate(
            [x_ref[...].astype(f32),
             xds[rows, 0:3 * NB].astype(f32), xu1], axis=1)
        acc = bg_ref[...].astype(f32)
        acc = acc + jnp.dot(xd, wgx_ref[...], preferred_element_type=f32)
        sterms = (s_ref[...].astype(f32), t0s[rows, :].astype(f32),
                  u0s[rows, :].astype(f32), t1s[rows, :].astype(f32), u1)
        for pos, t in enumerate(sterms):
            acc = acc + jnp.dot(t, wgs_ref[pos * SWID:(pos + 1) * SWID, :],
                                preferred_element_type=f32)
        zr = jax.nn.sigmoid(acc)
        z = zr[:, :SWID]
        rs[rows, :] = zr[:, SWID:]
        cbs[rows, :] = (z * s_ref[...]).astype(bf16)

    @pl.when(p == 4)
    def _():
        ab = a0s[rows, :].astype(bf16)
        t0s[rows, :] = (jnp.dot(ab, cbs[...], preferred_element_type=f32)
                        * INV).astype(bf16)

    @pl.when(p == 5)
    def _():
        ab = a0s[rows, :].astype(bf16)
        u0s[rows, :] = (jnp.dot(ab, t0s[...], preferred_element_type=f32)
                        * INV).astype(bf16)

    @pl.when(p == 6)
    def _():
        ab = a1s[rows, :].astype(bf16)
        t1s[rows, :] = (jnp.dot(ab, cbs[...], preferred_element_type=f32)
                        * INV).astype(bf16)

    @pl.when(p == 7)
    def _():
        ab = a1s[rows, :].astype(bf16)
        u1c = jnp.dot(ab, t1s[...], preferred_element_type=f32) * INV
        xd = jnp.concatenate(
            [x_ref[...].astype(f32), xds[rows, :].astype(f32)], axis=1)
        acc = bu_ref[...].astype(f32)
        acc = acc + jnp.dot(xd, wux_ref[...], preferred_element_type=f32)
        sterms = (cbs[rows, :].astype(f32), t0s[rows, :].astype(f32),
                  u0s[rows, :].astype(f32), t1s[rows, :].astype(f32), u1c)
        for pos, t in enumerate(sterms):
            acc = acc + jnp.dot(t, wus_ref[pos * SWID:(pos + 1) * SWID, :],
                                preferred_element_type=f32)
        hc = jnp.tanh(acc)
        r = rs[rows, :]
        h_ref[...] = r * s_ref[...] + (1.0 - r) * hc


def _mega(A0, A1, Yb, sT, xT, Wgx, Wgs, bg, Wux, Wus, bu):
    last = NBLK - 1
    return pl.pallas_call(
        _mega_kernel,
        grid=(8, NBLK),
        in_specs=[
            pl.BlockSpec((R, NODES),
                         lambda p, i: (jnp.where(p == 0, i, last), 0)),
            pl.BlockSpec((R, NODES),
                         lambda p, i: (jnp.where(p == 2, i,
                                                 jnp.where(p < 2, 0, last)), 0)),
            pl.BlockSpec((NODES, WID), lambda p, i: (0, 0)),
            pl.BlockSpec((R, SWID),
                         lambda p, i: (jnp.where((p == 3) | (p == 7), i, 0), 0)),
            pl.BlockSpec((R, NB),
                         lambda p, i: (jnp.where((p == 3) | (p == 7), i, 0), 0)),
            pl.BlockSpec((5 * NB, 2 * SWID), lambda p, i: (0, 0)),
            pl.BlockSpec((5 * SWID, 2 * SWID), lambda p, i: (0, 0)),
            pl.BlockSpec((1, 2 * SWID), lambda p, i: (0, 0)),
            pl.BlockSpec((5 * NB, SWID), lambda p, i: (0, 0)),
            pl.BlockSpec((5 * SWID, SWID), lambda p, i: (0, 0)),
            pl.BlockSpec((1, SWID), lambda p, i: (0, 0)),
        ],
        out_specs=pl.BlockSpec((R, SWID),
                               lambda p, i: (jnp.where(p == 7, i, 0), 0)),
        out_shape=jax.ShapeDtypeStruct((NODES, SWID), jnp.float32),
        scratch_shapes=[
            pltpu.VMEM((NODES, NODES), F8),
            pltpu.VMEM((NODES, NODES), F8),
            pltpu.VMEM((NODES, SWID), jnp.bfloat16),
            pltpu.VMEM((NODES, SWID), jnp.bfloat16),
            pltpu.VMEM((NODES, SWID), jnp.bfloat16),
            pltpu.VMEM((NODES, SWID), jnp.bfloat16),
            pltpu.VMEM((NODES, SWID), jnp.float32),
            pltpu.VMEM((NODES, 4 * NB), jnp.bfloat16),
        ],
    )(A0, A1, Yb, sT, xT, Wgx, Wgs, bg, Wux, Wus, bu)


def _expand_w(W5):
    O = W5.shape[-1]
    eye = jnp.eye(NB, dtype=W5.dtype)
    xpart = jnp.einsum('ib,po->pibo', eye, W5[:, 0, :])
    spart = jnp.einsum('bc,pho->pbhco', eye, W5[:, 1:, :])
    return (xpart.reshape(5 * NB, NB * O),
            spart.reshape(5 * NB * HID, NB * O))


def kernel(x, state, A0, A1, W_gate, b_gate, W_update, b_update):
    xT = x[:, :, 0].T
    sT = state.transpose(1, 0, 2).reshape(NODES, SWID)
    Yb = jnp.concatenate([xT, sT], axis=1).astype(jnp.bfloat16)

    W5g = W_gate.reshape(5, CIN, 2 * HID)
    Wzx, Wzs = _expand_w(W5g[:, :, :HID])
    Wrx, Wrs = _expand_w(W5g[:, :, HID:])
    Wgx = jnp.concatenate([Wzx, Wrx], axis=1)
    Wgs = jnp.concatenate([Wzs, Wrs], axis=1)
    bg = jnp.concatenate([jnp.tile(b_gate[:HID], NB),
                          jnp.tile(b_gate[HID:], NB)]).reshape(1, 2 * SWID)
    Wux, Wus = _expand_w(W_update.reshape(5, CIN, HID))
    bu = jnp.tile(b_update, NB).reshape(1, SWID)

    H = _mega(A0, A1, Yb, sT, xT, Wgx, Wgs, bg, Wux, Wus, bu)

    return H.reshape(NODES, NB, HID).transpose(1, 0, 2)


# speedup vs baseline: 1.3214x; 1.3214x over previous
import jax
import jax.numpy as jnp
from jax.experimental import pallas as pl
from jax.experimental.pallas import tpu as pltpu

NODES = 4096
HID = 32
NB = 4
CIN = HID + 1
WID = NB * CIN
SWID = NB * HID
RA = 128
RC = 512
NA = NODES // RA
NC = NODES // RC
B0, B1, B2, B3, B4, B5, B6, B7 = 32, 40, 72, 80, 88, 96, 104, 112
F8 = jnp.float8_e4m3fn
SCALE = 256.0
INV = 1.0 / SCALE


def _mega_kernel(a0_ref, a1_ref, y_ref, s_ref, x_ref, wgx_ref, wgs_ref,
                 bg_ref, wux_ref, wus_ref, bu_ref, h_ref,
                 a0s, a1s, t0s, t1s, u0s, cbs, rs, xds):
    s = pl.program_id(0)
    f32 = jnp.float32
    bf16 = jnp.bfloat16


    def first_pass(a_ref, as_scratch, ts_scratch, xd_col, lo):
        rows = pl.ds((s - lo) * RA, RA)
        a = a_ref[...]
        as_scratch[rows, :] = (a * SCALE).astype(F8)
        t = jnp.dot(a.astype(bf16), y_ref[...], preferred_element_type=f32)
        ts_scratch[rows, :] = t[:, NB:].astype(bf16)
        xds[rows, xd_col:xd_col + NB] = t[:, :NB].astype(bf16)

    @pl.when(s < B0)
    def _():
        first_pass(a0_ref, a0s, t0s, 0, 0)

    @pl.when((s >= B0) & (s < B1))
    def _():
        rows = pl.ds((s - B0) * RC, RC)
        ab = a0s[rows, :].astype(bf16)
        u0 = jnp.dot(ab, t0s[...], preferred_element_type=f32) * INV
        u0s[rows, :] = u0.astype(bf16)
        xu = jnp.dot(ab, xds[:, 0:NB], preferred_element_type=f32) * INV
        xds[rows, NB:2 * NB] = xu.astype(bf16)

    @pl.when((s >= B1) & (s < B2))
    def _():
        first_pass(a1_ref, a1s, t1s, 2 * NB, B1)

    @pl.when((s >= B2) & (s < B3))
    def _():
        rows = pl.ds((s - B2) * RC, RC)
        ab = a1s[rows, :].astype(bf16)
        u1 = jnp.dot(ab, t1s[...], preferred_element_type=f32) * INV
        xu1 = jnp.dot(ab, xds[:, 2 * NB:3 * NB],
                      preferred_element_type=f32) * INV
        xds[rows, 3 * NB:4 * NB] = xu1.astype(bf16)
        xd = jnp.concatenate(
            [x_ref[...].astype(f32),
             xds[rows, 0:3 * NB].astype(f32), xu1], axis=1)
        acc = bg_ref[...].astype(f32)
        acc = acc + jnp.dot(xd, wgx_ref[...], preferred_element_type=f32)
        sterms = (s_ref[...].astype(f32), t0s[rows, :].astype(f32),
                  u0s[rows, :].astype(f32), t1s[rows, :].astype(f32), u1)
        for pos, t in enumerate(sterms):
            acc = acc + jnp.dot(t, wgs_ref[pos * SWID:(pos + 1) * SWID, :],
                                preferred_element_type=f32)
        zr = jax.nn.sigmoid(acc)
        z = zr[:, :SWID]
        rs[rows, :] = zr[:, SWID:]
        cbs[rows, :] = (z * s_ref[...]).astype(bf16)

    @pl.when((s >= B3) & (s < B4))
    def _():
        rows = pl.ds((s - B3) * RC, RC)
        ab = a0s[rows, :].astype(bf16)
        t0s[rows, :] = (jnp.dot(ab, cbs[...], preferred_element_type=f32)
                        * INV).astype(bf16)

    @pl.when((s >= B4) & (s < B5))
    def _():
        rows = pl.ds((s - B4) * RC, RC)
        ab = a0s[rows, :].astype(bf16)
        u0s[rows, :] = (jnp.dot(ab, t0s[...], preferred_element_type=f32)
                        * INV).astype(bf16)

    @pl.when((s >= B5) & (s < B6))
    def _():
        rows = pl.ds((s - B5) * RC, RC)
        ab = a1s[rows, :].astype(bf16)
        t1s[rows, :] = (jnp.dot(ab, cbs[...], preferred_element_type=f32)
                        * INV).astype(bf16)

    @pl.when(s >= B6)
    def _():
        rows = pl.ds((s - B6) * RC, RC)
        ab = a1s[rows, :].astype(bf16)
        u1c = jnp.dot(ab, t1s[...], preferred_element_type=f32) * INV
        xd = jnp.concatenate(
            [x_ref[...].astype(f32), xds[rows, :].astype(f32)], axis=1)
        acc = bu_ref[...].astype(f32)
        acc = acc + jnp.dot(xd, wux_ref[...], preferred_element_type=f32)
        sterms = (cbs[rows, :].astype(f32), t0s[rows, :].astype(f32),
                  u0s[rows, :].astype(f32), t1s[rows, :].astype(f32), u1c)
        for pos, t in enumerate(sterms):
            acc = acc + jnp.dot(t, wus_ref[pos * SWID:(pos + 1) * SWID, :],
                                preferred_element_type=f32)
        hc = jnp.tanh(acc)
        r = rs[rows, :]
        h_ref[...] = r * s_ref[...] + (1.0 - r) * hc


def _mega(A0, A1, Yb, sT, xT, Wgx, Wgs, bg, Wux, Wus, bu):
    gate_or_final = lambda s: (s >= B2) & (s < B3) | (s >= B6)

    def rc_idx(s):
        return jnp.where((s >= B2) & (s < B3), s - B2,
                         jnp.where(s >= B6, s - B6, 0))

    return pl.pallas_call(
        _mega_kernel,
        grid=(B7,),
        in_specs=[
            pl.BlockSpec((RA, NODES),
                         lambda s: (jnp.where(s < B0, s, NA - 1), 0)),
            pl.BlockSpec((RA, NODES),
                         lambda s: (jnp.where((s >= B1) & (s < B2), s - B1,
                                              jnp.where(s < B1, 0, NA - 1)), 0)),
            pl.BlockSpec((NODES, WID), lambda s: (0, 0)),
            pl.BlockSpec((RC, SWID), lambda s: (rc_idx(s), 0)),
            pl.BlockSpec((RC, NB), lambda s: (rc_idx(s), 0)),
            pl.BlockSpec((5 * NB, 2 * SWID), lambda s: (0, 0)),
            pl.BlockSpec((5 * SWID, 2 * SWID), lambda s: (0, 0)),
            pl.BlockSpec((1, 2 * SWID), lambda s: (0, 0)),
            pl.BlockSpec((5 * NB, SWID), lambda s: (0, 0)),
            pl.BlockSpec((5 * SWID, SWID), lambda s: (0, 0)),
            pl.BlockSpec((1, SWID), lambda s: (0, 0)),
        ],
        out_specs=pl.BlockSpec((RC, SWID),
                               lambda s: (jnp.where(s >= B6, s - B6, 0), 0)),
        out_shape=jax.ShapeDtypeStruct((NODES, SWID), jnp.float32),
        scratch_shapes=[
            pltpu.VMEM((NODES, NODES), F8),
            pltpu.VMEM((NODES, NODES), F8),
            pltpu.VMEM((NODES, SWID), jnp.bfloat16),
            pltpu.VMEM((NODES, SWID), jnp.bfloat16),
            pltpu.VMEM((NODES, SWID), jnp.bfloat16),
            pltpu.VMEM((NODES, SWID), jnp.bfloat16),
            pltpu.VMEM((NODES, SWID), jnp.float32),
            pltpu.VMEM((NODES, 4 * NB), jnp.bfloat16),
        ],
    )(A0, A1, Yb, sT, xT, Wgx, Wgs, bg, Wux, Wus, bu)


def _expand_w(W5):
    O = W5.shape[-1]
    eye = jnp.eye(NB, dtype=W5.dtype)
    xpart = jnp.einsum('ib,po->pibo', eye, W5[:, 0, :])
    spart = jnp.einsum('bc,pho->pbhco', eye, W5[:, 1:, :])
    return (xpart.reshape(5 * NB, NB * O),
            spart.reshape(5 * NB * HID, NB * O))


def kernel(x, state, A0, A1, W_gate, b_gate, W_update, b_update):
    xT = x[:, :, 0].T
    sT = state.transpose(1, 0, 2).reshape(NODES, SWID)
    Yb = jnp.concatenate([xT, sT], axis=1).astype(jnp.bfloat16)

    W5g = W_gate.reshape(5, CIN, 2 * HID)
    Wzx, Wzs = _expand_w(W5g[:, :, :HID])
    Wrx, Wrs = _expand_w(W5g[:, :, HID:])
    Wgx = jnp.concatenate([Wzx, Wrx], axis=1)
    Wgs = jnp.concatenate([Wzs, Wrs], axis=1)
    bg = jnp.concatenate([jnp.tile(b_gate[:HID], NB),
                          jnp.tile(b_gate[HID:], NB)]).reshape(1, 2 * SWID)
    Wux, Wus = _expand_w(W_update.reshape(5, CIN, HID))
    bu = jnp.tile(b_update, NB).reshape(1, SWID)

    H = _mega(A0, A1, Yb, sT, xT, Wgx, Wgs, bg, Wux, Wus, bu)

    return H.reshape(NODES, NB, HID).transpose(1, 0, 2)


# speedup vs baseline: 1.6853x; 1.2753x over previous
import jax
import jax.numpy as jnp
from jax.experimental import pallas as pl
from jax.experimental.pallas import tpu as pltpu

NODES = 4096
HID = 32
NB = 4
CIN = HID + 1
WID = NB * CIN
SWID = NB * HID
RA = 128
RC = 512
NA = NODES // RA
NC = NODES // RC
B0, B1, B2, B3, B4, B5, B6, B7 = 32, 40, 72, 80, 88, 96, 104, 112
F8 = jnp.float8_e4m3fn
SA = 256.0
SI = 32.0
QA = 1.0 / (SA * SI)
ISI = 1.0 / SI


def _mega_kernel(a0_ref, a1_ref, y_ref, s_ref, x_ref, wgx_ref, wgs_ref,
                 bg_ref, wux_ref, wus_ref, bu_ref, h_ref,
                 a0s, a1s, t0s, t1s, u0s, cbs, rs, xds):
    s = pl.program_id(0)
    f32 = jnp.float32
    bf16 = jnp.bfloat16


    def f8dot(a8, b8):
        return jnp.dot(a8, b8, preferred_element_type=f32) * QA

    def store_i(ref_slice_setter, v):
        ref_slice_setter((v * SI).astype(F8))

    def first_pass(a_ref, as_scratch, ts_scratch, xd_col, lo):
        rows = pl.ds((s - lo) * RA, RA)
        a = a_ref[...]
        as_scratch[rows, :] = (a * SA).astype(F8)
        t = jnp.dot(a.astype(bf16), y_ref[...], preferred_element_type=f32)
        ts_scratch[rows, :] = (t[:, NB:] * SI).astype(F8)
        xds[rows, xd_col:xd_col + NB] = (t[:, :NB] * SI).astype(F8)

    @pl.when(s < B0)
    def _():
        first_pass(a0_ref, a0s, t0s, 0, 0)

    @pl.when((s >= B0) & (s < B1))
    def _():
        rows = pl.ds((s - B0) * RC, RC)
        ab = a0s[rows, :]
        u0 = f8dot(ab, t0s[...])
        u0s[rows, :] = (u0 * SI).astype(F8)
        xu = f8dot(ab, xds[:, 0:NB])
        xds[rows, NB:2 * NB] = (xu * SI).astype(F8)

    @pl.when((s >= B1) & (s < B2))
    def _():
        first_pass(a1_ref, a1s, t1s, 2 * NB, B1)

    @pl.when((s >= B2) & (s < B3))
    def _():
        rows = pl.ds((s - B2) * RC, RC)
        ab = a1s[rows, :]
        u1 = f8dot(ab, t1s[...])
        xu1 = f8dot(ab, xds[:, 2 * NB:3 * NB])
        xds[rows, 3 * NB:4 * NB] = (xu1 * SI).astype(F8)
        xd = jnp.concatenate(
            [x_ref[...].astype(f32),
             xds[rows, 0:3 * NB].astype(f32) * ISI, xu1], axis=1)
        acc = bg_ref[...].astype(f32)
        acc = acc + jnp.dot(xd, wgx_ref[...], preferred_element_type=f32)
        sterms = (s_ref[...].astype(f32),
                  t0s[rows, :].astype(f32) * ISI,
                  u0s[rows, :].astype(f32) * ISI,
                  t1s[rows, :].astype(f32) * ISI, u1)
        for pos, t in enumerate(sterms):
            acc = acc + jnp.dot(t, wgs_ref[pos * SWID:(pos + 1) * SWID, :],
                                preferred_element_type=f32)
        zr = jax.nn.sigmoid(acc)
        z = zr[:, :SWID]
        rs[rows, :] = zr[:, SWID:]
        cbs[rows, :] = (z * s_ref[...] * SI).astype(F8)

    @pl.when((s >= B3) & (s < B4))
    def _():
        rows = pl.ds((s - B3) * RC, RC)
        t0s[rows, :] = (f8dot(a0s[rows, :], cbs[...]) * SI).astype(F8)

    @pl.when((s >= B4) & (s < B5))
    def _():
        rows = pl.ds((s - B4) * RC, RC)
        u0s[rows, :] = (f8dot(a0s[rows, :], t0s[...]) * SI).astype(F8)

    @pl.when((s >= B5) & (s < B6))
    def _():
        rows = pl.ds((s - B5) * RC, RC)
        t1s[rows, :] = (f8dot(a1s[rows, :], cbs[...]) * SI).astype(F8)

    @pl.when(s >= B6)
    def _():
        rows = pl.ds((s - B6) * RC, RC)
        u1c = f8dot(a1s[rows, :], t1s[...])
        xd = jnp.concatenate(
            [x_ref[...].astype(f32), xds[rows, :].astype(f32) * ISI], axis=1)
        acc = bu_ref[...].astype(f32)
        acc = acc + jnp.dot(xd, wux_ref[...], preferred_element_type=f32)
        sterms = (cbs[rows, :].astype(f32) * ISI,
                  t0s[rows, :].astype(f32) * ISI,
                  u0s[rows, :].astype(f32) * ISI,
                  t1s[rows, :].astype(f32) * ISI, u1c)
        for pos, t in enumerate(sterms):
            acc = acc + jnp.dot(t, wus_ref[pos * SWID:(pos + 1) * SWID, :],
                                preferred_element_type=f32)
        hc = jnp.tanh(acc)
        r = rs[rows, :]
        h_ref[...] = r * s_ref[...] + (1.0 - r) * hc


def _mega(A0, A1, Yb, sT, xT, Wgx, Wgs, bg, Wux, Wus, bu):
    def rc_idx(s):
        return jnp.where((s >= B2) & (s < B3), s - B2,
                         jnp.where(s >= B6, s - B6, 0))

    return pl.pallas_call(
        _mega_kernel,
        grid=(B7,),
        in_specs=[
            pl.BlockSpec((RA, NODES),
                         lambda s: (jnp.where(s < B0, s, NA - 1), 0)),
            pl.BlockSpec((RA, NODES),
                         lambda s: (jnp.where((s >= B1) & (s < B2), s - B1,
                                              jnp.where(s < B1, 0, NA - 1)), 0)),
            pl.BlockSpec((NODES, WID), lambda s: (0, 0)),
            pl.BlockSpec((RC, SWID), lambda s: (rc_idx(s), 0)),
            pl.BlockSpec((RC, NB), lambda s: (rc_idx(s), 0)),
            pl.BlockSpec((5 * NB, 2 * SWID), lambda s: (0, 0)),
            pl.BlockSpec((5 * SWID, 2 * SWID), lambda s: (0, 0)),
            pl.BlockSpec((1, 2 * SWID), lambda s: (0, 0)),
            pl.BlockSpec((5 * NB, SWID), lambda s: (0, 0)),
            pl.BlockSpec((5 * SWID, SWID), lambda s: (0, 0)),
            pl.BlockSpec((1, SWID), lambda s: (0, 0)),
        ],
        out_specs=pl.BlockSpec((RC, SWID),
                               lambda s: (jnp.where(s >= B6, s - B6, 0), 0)),
        out_shape=jax.ShapeDtypeStruct((NODES, SWID), jnp.float32),
        scratch_shapes=[
            pltpu.VMEM((NODES, NODES), F8),
            pltpu.VMEM((NODES, NODES), F8),
            pltpu.VMEM((NODES, SWID), F8),
            pltpu.VMEM((NODES, SWID), F8),
            pltpu.VMEM((NODES, SWID), F8),
            pltpu.VMEM((NODES, SWID), F8),
            pltpu.VMEM((NODES, SWID), jnp.float32),
            pltpu.VMEM((NODES, 4 * NB), F8),
        ],
    )(A0, A1, Yb, sT, xT, Wgx, Wgs, bg, Wux, Wus, bu)


def _expand_w(W5):
    O = W5.shape[-1]
    eye = jnp.eye(NB, dtype=W5.dtype)
    xpart = jnp.einsum('ib,po->pibo', eye, W5[:, 0, :])
    spart = jnp.einsum('bc,pho->pbhco', eye, W5[:, 1:, :])
    return (xpart.reshape(5 * NB, NB * O),
            spart.reshape(5 * NB * HID, NB * O))


def kernel(x, state, A0, A1, W_gate, b_gate, W_update, b_update):
    xT = x[:, :, 0].T
    sT = state.transpose(1, 0, 2).reshape(NODES, SWID)
    Yb = jnp.concatenate([xT, sT], axis=1).astype(jnp.bfloat16)

    W5g = W_gate.reshape(5, CIN, 2 * HID)
    Wzx, Wzs = _expand_w(W5g[:, :, :HID])
    Wrx, Wrs = _expand_w(W5g[:, :, HID:])
    Wgx = jnp.concatenate([Wzx, Wrx], axis=1)
    Wgs = jnp.concatenate([Wzs, Wrs], axis=1)
    bg = jnp.concatenate([jnp.tile(b_gate[:HID], NB),
                          jnp.tile(b_gate[HID:], NB)]).reshape(1, 2 * SWID)
    Wux, Wus = _expand_w(W_update.reshape(5, CIN, HID))
    bu = jnp.tile(b_update, NB).reshape(1, SWID)

    H = _mega(A0, A1, Yb, sT, xT, Wgx, Wgs, bg, Wux, Wus, bu)

    return H.reshape(NODES, NB, HID).transpose(1, 0, 2)
